# baseline (speedup 1.0000x reference)
import jax
import jax.numpy as jnp
from jax import lax
from jax.experimental import pallas as pl
from jax.experimental.pallas import tpu as pltpu
from jax.experimental.pallas import tpu_sc as plsc

DEPTH = 1000
BATCH = 16384
NC = 2
NS = 16
L = 16
NW = NC * NS
SPW = BATCH // NW
CHUNK = 128
NCH = SPW // CHUNK
NG = CHUNK // L


def _body(x_hbm, out_hbm, idx_v, buf, sem):
    wid = lax.axis_index("s") * NC + lax.axis_index("c")
    base = wid * SPW
    pltpu.sync_copy(x_hbm.at[pl.ds(base, SPW)], idx_v)

    zeros = jnp.zeros((L,), jnp.float32)
    ones = jnp.full((L,), 1.0, jnp.float32)
    lane = lax.iota(jnp.int32, L)

    def zfill(i, carry):
        r0 = i * 8
        for dr in range(8):
            for j in range(CHUNK // L):
                buf[r0 + dr, pl.ds(j * L, L)] = zeros
        return carry

    lax.fori_loop(0, DEPTH // 8, zfill, 0)

    def scatter(ch, val):
        for g in range(NG):
            row = idx_v[pl.ds(ch * CHUNK + g * L, L)]
            col = lane + (g * L)
            plsc.store_scatter(buf, [row, col], val)

    for ch in range(NCH):
        scatter(ch, ones)
        col0 = base + ch * CHUNK
        cp = pltpu.make_async_copy(
            buf, out_hbm.at[:, pl.ds(col0, CHUNK)], sem)
        cp.start()
        cp.wait()
        if ch < NCH - 1:
            scatter(ch, zeros)


@jax.jit
def _onehot_t(x):
    mesh = plsc.VectorSubcoreMesh(core_axis_name="c", subcore_axis_name="s")
    k = pl.kernel(
        _body,
        out_type=jax.ShapeDtypeStruct((DEPTH, BATCH), jnp.float32),
        mesh=mesh,
        scratch_types=[
            pltpu.VMEM((SPW,), jnp.int32),
            pltpu.VMEM((DEPTH, CHUNK), jnp.float32),
            pltpu.SemaphoreType.DMA,
        ],
        compiler_params=pltpu.CompilerParams(
            needs_layout_passes=False,
            use_tc_tiling_on_sc=True,
        ),
    )
    return k(x)


def kernel(X_in, ones):
    return _onehot_t(X_in.astype(jnp.int32)).T


# speedup vs baseline: 1.0192x; 1.0192x over previous
import jax
import jax.numpy as jnp
from jax import lax
from jax.experimental import pallas as pl
from jax.experimental.pallas import tpu as pltpu
from jax.experimental.pallas import tpu_sc as plsc

DEPTH = 1000
BATCH = 16384
NC = 2
NS = 16
L = 16
NW = NC * NS
SPW = BATCH // NW
CHUNK = 128
NCH = SPW // CHUNK
NG = CHUNK // L


def _body(x_hbm, out_hbm, idx_v, buf, sem):
    wid = lax.axis_index("s") * NC + lax.axis_index("c")
    base = wid * SPW
    pltpu.sync_copy(x_hbm.at[pl.ds(base, SPW)], idx_v)

    zeros = jnp.zeros((L,), jnp.float32)
    ones = jnp.full((L,), 1.0, jnp.float32)
    lane = lax.iota(jnp.int32, L)

    def zbody(r, carry):
        for j in range(CHUNK // L):
            buf[r, pl.ds(j * L, L)] = zeros
        return carry

    lax.fori_loop(0, DEPTH, zbody, 0)

    def scatter(ch, val):
        def g_body(g, carry):
            row = idx_v[pl.ds(ch * CHUNK + g * L, L)]
            col = lane + g * L
            plsc.store_scatter(buf, [row, col], val)
            return carry
        lax.fori_loop(0, NG, g_body, 0)

    def ch_body(ch, carry):
        scatter(ch, ones)
        col0 = base + ch * CHUNK
        cp = pltpu.make_async_copy(
            buf, out_hbm.at[:, pl.ds(col0, CHUNK)], sem)
        cp.start()
        cp.wait()
        scatter(ch, zeros)
        return carry

    lax.fori_loop(0, NCH, ch_body, 0)


@jax.jit
def _onehot_t(x):
    mesh = plsc.VectorSubcoreMesh(core_axis_name="c", subcore_axis_name="s")
    k = pl.kernel(
        _body,
        out_type=jax.ShapeDtypeStruct((DEPTH, BATCH), jnp.float32),
        mesh=mesh,
        scratch_types=[
            pltpu.VMEM((SPW,), jnp.int32),
            pltpu.VMEM((DEPTH, CHUNK), jnp.float32),
            pltpu.SemaphoreType.DMA,
        ],
        compiler_params=pltpu.CompilerParams(
            needs_layout_passes=False,
            use_tc_tiling_on_sc=True,
        ),
    )
    return k(x)


def kernel(X_in, ones):
    return _onehot_t(X_in.astype(jnp.int32)).T


# speedup vs baseline: 1.0734x; 1.0531x over previous
import jax
import jax.numpy as jnp
from jax import lax
from jax.experimental import pallas as pl
from jax.experimental.pallas import tpu as pltpu
from jax.experimental.pallas import tpu_sc as plsc

DEPTH = 1000
BATCH = 16384
NC = 2
NS = 16
L = 16
NW = NC * NS
SPW = BATCH // NW
CHUNK = 128
NCH = SPW // CHUNK
NG = CHUNK // L
TOP = 496
BOT = DEPTH - TOP


def _body(x_hbm, out_hbm, idx_v, buf, sem_t, sem_b, sem_i):
    wid = lax.axis_index("s") * NC + lax.axis_index("c")
    base = wid * SPW
    cp_idx = pltpu.make_async_copy(x_hbm.at[pl.ds(base, SPW)], idx_v, sem_i)
    cp_idx.start()

    zeros = jnp.zeros((L,), jnp.float32)
    ones = jnp.full((L,), 1.0, jnp.float32)
    lane = lax.iota(jnp.int32, L)
    topv = jnp.full((L,), TOP, jnp.int32)

    def zfill(r0, r1):
        def zbody(r, carry):
            for j in range(CHUNK // L):
                buf[r, pl.ds(j * L, L)] = zeros
            return carry
        lax.fori_loop(r0, r1, zbody, 0)

    def scatter(ch, val, top_half):
        def g_body(g, carry):
            row = idx_v[pl.ds(ch * CHUNK + g * L, L)]
            col = lane + g * L
            mask = (row < topv) if top_half else (row >= topv)
            plsc.store_scatter(buf, [row, col], val, mask=mask)
            return carry
        lax.fori_loop(0, NG, g_body, 0)

    def dma(ch, top_half):
        col0 = base + ch * CHUNK
        if top_half:
            return pltpu.make_async_copy(
                buf.at[pl.ds(0, TOP), :],
                out_hbm.at[pl.ds(0, TOP), pl.ds(col0, CHUNK)], sem_t)
        return pltpu.make_async_copy(
            buf.at[pl.ds(TOP, BOT), :],
            out_hbm.at[pl.ds(TOP, BOT), pl.ds(col0, CHUNK)], sem_b)

    zfill(0, TOP)
    cp_idx.wait()
    scatter(0, ones, True)
    dma(0, True).start()
    zfill(TOP, DEPTH)
    scatter(0, ones, False)
    dma(0, False).start()

    def ch_body(ch, carry):
        dma(ch - 1, True).wait()
        scatter(ch - 1, zeros, True)
        scatter(ch, ones, True)
        dma(ch, True).start()
        dma(ch - 1, False).wait()
        scatter(ch - 1, zeros, False)
        scatter(ch, ones, False)
        dma(ch, False).start()
        return carry

    lax.fori_loop(1, NCH, ch_body, 0)
    dma(NCH - 1, True).wait()
    dma(NCH - 1, False).wait()


@jax.jit
def _onehot_t(x):
    mesh = plsc.VectorSubcoreMesh(core_axis_name="c", subcore_axis_name="s")
    k = pl.kernel(
        _body,
        out_type=jax.ShapeDtypeStruct((DEPTH, BATCH), jnp.float32),
        mesh=mesh,
        scratch_types=[
            pltpu.VMEM((SPW,), jnp.int32),
            pltpu.VMEM((DEPTH, CHUNK), jnp.float32),
            pltpu.SemaphoreType.DMA,
            pltpu.SemaphoreType.DMA,
            pltpu.SemaphoreType.DMA,
        ],
        compiler_params=pltpu.CompilerParams(
            needs_layout_passes=False,
            use_tc_tiling_on_sc=True,
        ),
    )
    return k(x)


def kernel(X_in, ones):
    return _onehot_t(X_in.astype(jnp.int32)).T


# speedup vs baseline: 1.0913x; 1.0167x over previous
import jax
import jax.numpy as jnp
from jax import lax
from jax.experimental import pallas as pl
from jax.experimental.pallas import tpu as pltpu
from jax.experimental.pallas import tpu_sc as plsc

DEPTH = 1000
BATCH = 16384
NC = 2
NS = 16
L = 16
NW = NC * NS
SPW = BATCH // NW
CHUNK = 128
NCH = SPW // CHUNK
NG = CHUNK // L
BANDS = ((0, 248), (248, 496), (496, 744), (744, 1000))


def _body(x_hbm, out_hbm, idx_v, buf, sem0, sem1, sem2, sem3, sem_i):
    wid = lax.axis_index("s") * NC + lax.axis_index("c")
    base = wid * SPW
    cp_idx = pltpu.make_async_copy(x_hbm.at[pl.ds(base, SPW)], idx_v, sem_i)
    cp_idx.start()

    sems = (sem0, sem1, sem2, sem3)
    zeros = jnp.zeros((L,), jnp.float32)
    ones = jnp.full((L,), 1.0, jnp.float32)
    lane = lax.iota(jnp.int32, L)

    def zfill(r0, r1):
        def zbody(r, carry):
            for j in range(CHUNK // L):
                buf[r, pl.ds(j * L, L)] = zeros
            return carry
        lax.fori_loop(r0, r1, zbody, 0)

    def scatter(ch, val, b):
        lo, hi = BANDS[b]
        lov = jnp.full((L,), lo, jnp.int32)
        hiv = jnp.full((L,), hi, jnp.int32)

        def g_body(g, carry):
            row = idx_v[pl.ds(ch * CHUNK + g * L, L)]
            col = lane + g * L
            mask = (row >= lov) & (row < hiv) if lo else (row < hiv)
            plsc.store_scatter(buf, [row, col], val, mask=mask)
            return carry
        lax.fori_loop(0, NG, g_body, 0)

    def dma(ch, b):
        lo, hi = BANDS[b]
        col0 = base + ch * CHUNK
        return pltpu.make_async_copy(
            buf.at[pl.ds(lo, hi - lo), :],
            out_hbm.at[pl.ds(lo, hi - lo), pl.ds(col0, CHUNK)], sems[b])

    zfill(*BANDS[0])
    cp_idx.wait()
    scatter(0, ones, 0)
    dma(0, 0).start()
    for b in (1, 2, 3):
        zfill(*BANDS[b])
        scatter(0, ones, b)
        dma(0, b).start()

    def ch_body(ch, carry):
        for b in range(4):
            dma(ch - 1, b).wait()
            scatter(ch - 1, zeros, b)
            scatter(ch, ones, b)
            dma(ch, b).start()
        return carry

    lax.fori_loop(1, NCH, ch_body, 0)
    for b in range(4):
        dma(NCH - 1, b).wait()


@jax.jit
def _onehot_t(x):
    mesh = plsc.VectorSubcoreMesh(core_axis_name="c", subcore_axis_name="s")
    k = pl.kernel(
        _body,
        out_type=jax.ShapeDtypeStruct((DEPTH, BATCH), jnp.float32),
        mesh=mesh,
        scratch_types=[
            pltpu.VMEM((SPW,), jnp.int32),
            pltpu.VMEM((DEPTH, CHUNK), jnp.float32),
            pltpu.SemaphoreType.DMA,
            pltpu.SemaphoreType.DMA,
            pltpu.SemaphoreType.DMA,
            pltpu.SemaphoreType.DMA,
            pltpu.SemaphoreType.DMA,
        ],
        compiler_params=pltpu.CompilerParams(
            needs_layout_passes=False,
            use_tc_tiling_on_sc=True,
        ),
    )
    return k(x)


def kernel(X_in, ones):
    return _onehot_t(X_in.astype(jnp.int32)).T
